# baseline (speedup 1.0000x reference)
import functools

import jax
import jax.numpy as jnp
import numpy as np
from jax import lax
from jax.experimental import pallas as pl
from jax.experimental.pallas import tpu as pltpu
from jax.experimental.pallas import tpu_sc as plsc

D_MODEL = 768
INTER = 2048
E = 8
T = 2048
A = 2 * T
G = 128
PT = 5120
NB = PT // G

_TRI = np.tril(np.ones((256, 256), np.float32), -1)
_GM = np.kron(np.eye(E, dtype=np.float32),
              np.triu(np.ones((16, 16), np.float32), 1))
_SEL = np.tile(np.eye(16, dtype=np.float32), (E, 1))


def _router_body(x_ref, rw_ref, out_ref):
    x = x_ref[...]
    logits = lax.dot_general(x, rw_ref[...], (((1,), (1,)), ((), ())),
                             preferred_element_type=jnp.float32)
    idx = lax.broadcasted_iota(jnp.int32, (T, E), 1)
    m1 = jnp.max(logits, axis=1, keepdims=True)
    e1 = jnp.min(jnp.where(logits == m1, idx, E), axis=1, keepdims=True)
    masked = jnp.where(idx == e1, -jnp.inf, logits)
    m2 = jnp.max(masked, axis=1, keepdims=True)
    e2 = jnp.min(jnp.where(masked == m2, idx, E), axis=1, keepdims=True)
    r = jnp.exp(m2 - m1)
    w1 = 1.0 / (1.0 + r)
    w2 = 1.0 - w1
    out_ref[...] = jnp.concatenate(
        [e1.astype(jnp.float32), e2.astype(jnp.float32), w1, w2], axis=1)


def _rank_body(eft_ref, tri_ref, gm_ref, sel_ref, rank_ref, tot_ref):
    eft = eft_ref[...]
    ef_rep = jnp.concatenate([eft] * E, axis=1)
    e_iota = lax.broadcasted_iota(jnp.int32, (256, 128), 1) // 16
    oh = (ef_rep == e_iota).astype(jnp.float32)
    within = lax.dot_general(tri_ref[...], oh, (((1,), (0,)), ((), ())),
                             preferred_element_type=jnp.float32)
    tot = jnp.sum(oh, axis=0, keepdims=True)
    prefix = lax.dot_general(tot, gm_ref[...], (((1,), (0,)), ((), ())),
                             preferred_element_type=jnp.float32)
    rank_full = (within + prefix) * oh
    rank_ref[...] = lax.dot_general(rank_full, sel_ref[...],
                                    (((1,), (0,)), ((), ())),
                                    precision=lax.Precision.HIGHEST,
                                    preferred_element_type=jnp.float32)
    tot_ref[...] = jnp.broadcast_to(tot, (8, 128))


def _ffn_body(be_ref, xs_ref, gw_ref, uw_ref, dw_ref, ws_ref, out_ref):
    del be_ref
    x = xs_ref[...].astype(jnp.bfloat16)
    g = lax.dot_general(x, gw_ref[0], (((1,), (1,)), ((), ())),
                        preferred_element_type=jnp.float32)
    u = lax.dot_general(x, uw_ref[0], (((1,), (1,)), ((), ())),
                        preferred_element_type=jnp.float32)
    h = (g * lax.logistic(g) * u).astype(jnp.bfloat16)
    y = lax.dot_general(h, dw_ref[0], (((1,), (1,)), ((), ())),
                        preferred_element_type=jnp.float32)
    out_ref[...] = y * ws_ref[...]


@functools.lru_cache(maxsize=1)
def _sc_kernels():
    mesh = plsc.VectorSubcoreMesh(core_axis_name="c", subcore_axis_name="s")
    nc = mesh.num_cores
    nw = nc * mesh.num_subcores

    @functools.partial(
        pl.kernel,
        out_type=(jax.ShapeDtypeStruct((PT,), jnp.int32),
                  jax.ShapeDtypeStruct((PT,), jnp.float32),
                  jax.ShapeDtypeStruct((A,), jnp.int32)),
        mesh=mesh,
        compiler_params=pltpu.CompilerParams(needs_layout_passes=False),
        scratch_types=[pltpu.VMEM((A,), jnp.int32),
                       pltpu.VMEM((A,), jnp.int32),
                       pltpu.VMEM((16,), jnp.int32),
                       pltpu.VMEM((A,), jnp.float32),
                       pltpu.VMEM((PT,), jnp.int32),
                       pltpu.VMEM((PT,), jnp.float32),
                       pltpu.VMEM((A,), jnp.int32)])
    def dispatch(ef_hbm, rank_hbm, pb_hbm, wf_hbm, zi_hbm, zf_hbm,
                 order_hbm, ws_hbm, dest_hbm,
                 ef_v, rank_v, pb_v, wf_v, order_v, ws_v, dest_v):
        wid = lax.axis_index("s") * nc + lax.axis_index("c")

        @pl.when(wid == 0)
        def _():
            pltpu.sync_copy(ef_hbm, ef_v)
            pltpu.sync_copy(rank_hbm, rank_v)
            pltpu.sync_copy(pb_hbm, pb_v)
            pltpu.sync_copy(wf_hbm, wf_v)
            pltpu.sync_copy(zi_hbm, order_v)
            pltpu.sync_copy(zf_hbm, ws_v)

            def body(i, carry):
                sl = pl.ds(i * 16, 16)
                efv = ef_v[sl]
                base = plsc.load_gather(pb_v, [efv])
                dv = base + rank_v[sl]
                tok = (lax.iota(jnp.int32, 16) + i * 16) & (T - 1)
                plsc.store_scatter(order_v, [dv], tok)
                plsc.store_scatter(ws_v, [dv], wf_v[sl])
                dest_v[sl] = dv
                return carry

            lax.fori_loop(0, A // 16, body, 0)
            pltpu.sync_copy(order_v, order_hbm)
            pltpu.sync_copy(ws_v, ws_hbm)
            pltpu.sync_copy(dest_v, dest_hbm)

    RW = PT // 32
    CH = 40
    NCH = RW // CH

    @functools.partial(
        pl.kernel,
        out_type=jax.ShapeDtypeStruct((PT, D_MODEL), jnp.float32),
        mesh=mesh,
        scratch_types=[pltpu.VMEM((RW,), jnp.int32),
                       pltpu.VMEM((CH, D_MODEL), jnp.float32),
                       pltpu.VMEM((CH, D_MODEL), jnp.float32),
                       pltpu.SemaphoreType.DMA,
                       pltpu.SemaphoreType.DMA,
                       pltpu.SemaphoreType.DMA,
                       pltpu.SemaphoreType.DMA])
    def gather_rows(x_hbm, order_hbm, xs_hbm, idx_v, r0, r1, g0, g1, o0, o1):
        wid = lax.axis_index("s") * nc + lax.axis_index("c")
        base = wid * RW
        pltpu.sync_copy(order_hbm.at[pl.ds(base, RW)], idx_v)
        bufs = (r0, r1)
        gsem = (g0, g1)
        osem = (o0, o1)
        gathers = []
        outs = [None, None]
        for c in range(NCH):
            p = c % 2
            if c >= 2:
                outs[p].wait()
            gathers.append(pltpu.async_copy(
                x_hbm.at[idx_v.at[pl.ds(c * CH, CH)]], bufs[p], gsem[p]))
            if c >= 1:
                gathers[c - 1].wait()
                outs[(c - 1) % 2] = pltpu.async_copy(
                    bufs[(c - 1) % 2], xs_hbm.at[pl.ds(base + (c - 1) * CH, CH)],
                    osem[(c - 1) % 2])
        gathers[NCH - 1].wait()
        if NCH >= 2:
            outs[(NCH - 2) % 2].wait()
        pltpu.sync_copy(bufs[(NCH - 1) % 2],
                        xs_hbm.at[pl.ds(base + (NCH - 1) * CH, CH)])

    @functools.partial(
        pl.kernel,
        out_type=jax.ShapeDtypeStruct((T, D_MODEL), jnp.float32),
        mesh=mesh,
        scratch_types=[pltpu.VMEM((64,), jnp.int32),
                       pltpu.VMEM((64,), jnp.int32),
                       pltpu.VMEM((64, D_MODEL), jnp.float32),
                       pltpu.VMEM((64, D_MODEL), jnp.float32),
                       pltpu.SemaphoreType.DMA,
                       pltpu.SemaphoreType.DMA])
    def combine(y_hbm, d1_hbm, d2_hbm, o_hbm, i1_v, i2_v, y1_v, y2_v, s1, s2):
        wid = lax.axis_index("s") * nc + lax.axis_index("c")
        base = wid * (T // nw)
        pltpu.sync_copy(d1_hbm.at[pl.ds(base, 64)], i1_v)
        pltpu.sync_copy(d2_hbm.at[pl.ds(base, 64)], i2_v)
        cp1 = pltpu.async_copy(y_hbm.at[i1_v], y1_v, s1)
        cp2 = pltpu.async_copy(y_hbm.at[i2_v], y2_v, s2)
        cp1.wait()
        cp2.wait()

        def row(r, carry):
            for cc in range(D_MODEL // 16):
                sl = pl.ds(cc * 16, 16)
                y1_v[r, sl] = y1_v[r, sl] + y2_v[r, sl]
            return carry

        lax.fori_loop(0, 64, row, 0)
        pltpu.sync_copy(y1_v, o_hbm.at[pl.ds(base, 64)])

    return dispatch, gather_rows, combine


def kernel(hidden_states, router_w, gate_w, up_w, down_w):
    bsz, seq_len, d_model = hidden_states.shape
    x = hidden_states.reshape(-1, d_model)

    sched = pl.pallas_call(
        _router_body,
        out_shape=jax.ShapeDtypeStruct((T, 4), jnp.float32),
    )(x, router_w)
    e1 = sched[:, 0].astype(jnp.int32)
    e2 = sched[:, 1].astype(jnp.int32)
    ef = jnp.concatenate([e1, e2])
    wf = jnp.concatenate([sched[:, 2], sched[:, 3]])

    ef_t = ef.reshape(16, 256).T
    rank_t, totb = pl.pallas_call(
        _rank_body,
        out_shape=(jax.ShapeDtypeStruct((256, 16), jnp.float32),
                   jax.ShapeDtypeStruct((8, 128), jnp.float32)),
    )(ef_t, jnp.asarray(_TRI), jnp.asarray(_GM), jnp.asarray(_SEL))
    rank = rank_t.T.reshape(A).astype(jnp.int32)
    counts = totb[0].reshape(E, 16).sum(axis=1).astype(jnp.int32)

    nblk = (counts + G - 1) // G
    ends = jnp.cumsum(nblk)
    starts = jnp.concatenate([jnp.zeros(1, jnp.int32), ends[:-1]])
    pb16 = jnp.concatenate([starts * G, jnp.zeros(8, jnp.int32)])
    bid = jnp.arange(NB, dtype=jnp.int32)
    block_expert = jnp.minimum(
        jnp.sum((bid[:, None] >= ends[None, :]).astype(jnp.int32), axis=1),
        E - 1).astype(jnp.int32)

    dispatch, gather_rows, combine = _sc_kernels()
    order, ws, dest = dispatch(ef, rank, pb16, wf,
                               jnp.zeros((PT,), jnp.int32),
                               jnp.zeros((PT,), jnp.float32))

    xs = gather_rows(x, order)

    y = pl.pallas_call(
        _ffn_body,
        grid_spec=pltpu.PrefetchScalarGridSpec(
            num_scalar_prefetch=1,
            grid=(NB,),
            in_specs=[
                pl.BlockSpec((G, D_MODEL), lambda b, be: (b, 0)),
                pl.BlockSpec((1, INTER, D_MODEL), lambda b, be: (be[b], 0, 0)),
                pl.BlockSpec((1, INTER, D_MODEL), lambda b, be: (be[b], 0, 0)),
                pl.BlockSpec((1, D_MODEL, INTER), lambda b, be: (be[b], 0, 0)),
                pl.BlockSpec((G, 1), lambda b, be: (b, 0)),
            ],
            out_specs=pl.BlockSpec((G, D_MODEL), lambda b, be: (b, 0)),
        ),
        out_shape=jax.ShapeDtypeStruct((PT, D_MODEL), jnp.float32),
        compiler_params=pltpu.CompilerParams(
            dimension_semantics=("arbitrary",)),
    )(block_expert, xs, gate_w.astype(jnp.bfloat16),
      up_w.astype(jnp.bfloat16), down_w.astype(jnp.bfloat16),
      ws.reshape(PT, 1))

    out = combine(y, dest[:T], dest[T:])
    return out.reshape(bsz, seq_len, d_model)


# speedup vs baseline: 1.2175x; 1.2175x over previous
import functools

import jax
import jax.numpy as jnp
import numpy as np
from jax import lax
from jax.experimental import pallas as pl
from jax.experimental.pallas import tpu as pltpu
from jax.experimental.pallas import tpu_sc as plsc

D_MODEL = 768
INTER = 2048
E = 8
T = 2048
A = 2 * T
G = 256
PT = 6144
NB = PT // G

_TRI = np.tril(np.ones((256, 256), np.float32), -1)
_GM = np.kron(np.eye(E, dtype=np.float32),
              np.triu(np.ones((16, 16), np.float32), 1))
_SEL = np.tile(np.eye(16, dtype=np.float32), (E, 1))


def _router_body(x_ref, rw_ref, out_ref):
    x = x_ref[...]
    logits = lax.dot_general(x, rw_ref[...], (((1,), (1,)), ((), ())),
                             preferred_element_type=jnp.float32)
    idx = lax.broadcasted_iota(jnp.int32, (T, E), 1)
    m1 = jnp.max(logits, axis=1, keepdims=True)
    e1 = jnp.min(jnp.where(logits == m1, idx, E), axis=1, keepdims=True)
    masked = jnp.where(idx == e1, -jnp.inf, logits)
    m2 = jnp.max(masked, axis=1, keepdims=True)
    e2 = jnp.min(jnp.where(masked == m2, idx, E), axis=1, keepdims=True)
    r = jnp.exp(m2 - m1)
    w1 = 1.0 / (1.0 + r)
    w2 = 1.0 - w1
    out_ref[...] = jnp.concatenate(
        [e1.astype(jnp.float32), e2.astype(jnp.float32), w1, w2], axis=1)


def _rank_body(eft_ref, tri_ref, gm_ref, sel_ref, rank_ref, tot_ref):
    eft = eft_ref[...]
    ef_rep = jnp.concatenate([eft] * E, axis=1)
    e_iota = lax.broadcasted_iota(jnp.int32, (256, 128), 1) // 16
    oh = (ef_rep == e_iota).astype(jnp.float32)
    within = lax.dot_general(tri_ref[...], oh, (((1,), (0,)), ((), ())),
                             preferred_element_type=jnp.float32)
    tot = jnp.sum(oh, axis=0, keepdims=True)
    prefix = lax.dot_general(tot, gm_ref[...], (((1,), (0,)), ((), ())),
                             preferred_element_type=jnp.float32)
    rank_full = (within + prefix) * oh
    rank_ref[...] = lax.dot_general(rank_full, sel_ref[...],
                                    (((1,), (0,)), ((), ())),
                                    precision=lax.Precision.HIGHEST,
                                    preferred_element_type=jnp.float32)
    tot_ref[...] = jnp.broadcast_to(tot, (8, 128))


def _ffn_body(be_ref, xs_ref, gw_ref, uw_ref, dw_ref, ws_ref, out_ref):
    del be_ref
    x = xs_ref[...]
    g = lax.dot_general(x, gw_ref[0], (((1,), (1,)), ((), ())),
                        preferred_element_type=jnp.float32)
    u = lax.dot_general(x, uw_ref[0], (((1,), (1,)), ((), ())),
                        preferred_element_type=jnp.float32)
    h = g * lax.logistic(g) * u
    y = lax.dot_general(h, dw_ref[0], (((1,), (1,)), ((), ())),
                        preferred_element_type=jnp.float32)
    out_ref[...] = y * ws_ref[...]


@functools.lru_cache(maxsize=1)
def _sc_kernels():
    mesh = plsc.VectorSubcoreMesh(core_axis_name="c", subcore_axis_name="s")
    nc = mesh.num_cores
    nw = nc * mesh.num_subcores

    @functools.partial(
        pl.kernel,
        out_type=(jax.ShapeDtypeStruct((PT,), jnp.int32),
                  jax.ShapeDtypeStruct((PT,), jnp.float32),
                  jax.ShapeDtypeStruct((A,), jnp.int32)),
        mesh=mesh,
        compiler_params=pltpu.CompilerParams(needs_layout_passes=False),
        scratch_types=[pltpu.VMEM((A,), jnp.int32),
                       pltpu.VMEM((A,), jnp.int32),
                       pltpu.VMEM((16,), jnp.int32),
                       pltpu.VMEM((A,), jnp.float32),
                       pltpu.VMEM((PT,), jnp.int32),
                       pltpu.VMEM((PT,), jnp.float32),
                       pltpu.VMEM((A,), jnp.int32)])
    def dispatch(ef_hbm, rank_hbm, pb_hbm, wf_hbm, zi_hbm, zf_hbm,
                 order_hbm, ws_hbm, dest_hbm,
                 ef_v, rank_v, pb_v, wf_v, order_v, ws_v, dest_v):
        wid = lax.axis_index("s") * nc + lax.axis_index("c")

        @pl.when(wid == 0)
        def _():
            pltpu.sync_copy(ef_hbm, ef_v)
            pltpu.sync_copy(rank_hbm, rank_v)
            pltpu.sync_copy(pb_hbm, pb_v)
            pltpu.sync_copy(wf_hbm, wf_v)
            pltpu.sync_copy(zi_hbm, order_v)
            pltpu.sync_copy(zf_hbm, ws_v)

            def body(i, carry):
                sl = pl.ds(i * 16, 16)
                efv = ef_v[sl]
                base = plsc.load_gather(pb_v, [efv])
                dv = base + rank_v[sl]
                tok = (lax.iota(jnp.int32, 16) + i * 16) & (T - 1)
                plsc.store_scatter(order_v, [dv], tok)
                plsc.store_scatter(ws_v, [dv], wf_v[sl])
                dest_v[sl] = dv
                return carry

            lax.fori_loop(0, A // 16, body, 0)
            pltpu.sync_copy(order_v, order_hbm)
            pltpu.sync_copy(ws_v, ws_hbm)
            pltpu.sync_copy(dest_v, dest_hbm)

    RW = PT // 32
    CH = 48
    NCH = RW // CH

    @functools.partial(
        pl.kernel,
        out_type=jax.ShapeDtypeStruct((PT, D_MODEL), jnp.float32),
        mesh=mesh,
        scratch_types=[pltpu.VMEM((RW,), jnp.int32),
                       pltpu.VMEM((CH, D_MODEL), jnp.float32),
                       pltpu.VMEM((CH, D_MODEL), jnp.float32),
                       pltpu.SemaphoreType.DMA,
                       pltpu.SemaphoreType.DMA,
                       pltpu.SemaphoreType.DMA,
                       pltpu.SemaphoreType.DMA])
    def gather_rows(x_hbm, order_hbm, xs_hbm, idx_v, r0, r1, g0, g1, o0, o1):
        wid = lax.axis_index("s") * nc + lax.axis_index("c")
        base = wid * RW
        pltpu.sync_copy(order_hbm.at[pl.ds(base, RW)], idx_v)
        bufs = (r0, r1)
        gsem = (g0, g1)
        osem = (o0, o1)
        gathers = []
        outs = [None, None]
        for c in range(NCH):
            p = c % 2
            if c >= 2:
                outs[p].wait()
            gathers.append(pltpu.async_copy(
                x_hbm.at[idx_v.at[pl.ds(c * CH, CH)]], bufs[p], gsem[p]))
            if c >= 1:
                gathers[c - 1].wait()
                outs[(c - 1) % 2] = pltpu.async_copy(
                    bufs[(c - 1) % 2], xs_hbm.at[pl.ds(base + (c - 1) * CH, CH)],
                    osem[(c - 1) % 2])
        gathers[NCH - 1].wait()
        if NCH >= 2:
            outs[(NCH - 2) % 2].wait()
        pltpu.sync_copy(bufs[(NCH - 1) % 2],
                        xs_hbm.at[pl.ds(base + (NCH - 1) * CH, CH)])

    @functools.partial(
        pl.kernel,
        out_type=jax.ShapeDtypeStruct((T, D_MODEL), jnp.float32),
        mesh=mesh,
        scratch_types=[pltpu.VMEM((64,), jnp.int32),
                       pltpu.VMEM((64,), jnp.int32),
                       pltpu.VMEM((64, D_MODEL), jnp.float32),
                       pltpu.VMEM((64, D_MODEL), jnp.float32),
                       pltpu.SemaphoreType.DMA,
                       pltpu.SemaphoreType.DMA])
    def combine(y_hbm, d1_hbm, d2_hbm, o_hbm, i1_v, i2_v, y1_v, y2_v, s1, s2):
        wid = lax.axis_index("s") * nc + lax.axis_index("c")
        base = wid * (T // nw)
        pltpu.sync_copy(d1_hbm.at[pl.ds(base, 64)], i1_v)
        pltpu.sync_copy(d2_hbm.at[pl.ds(base, 64)], i2_v)
        cp1 = pltpu.async_copy(y_hbm.at[i1_v], y1_v, s1)
        cp2 = pltpu.async_copy(y_hbm.at[i2_v], y2_v, s2)
        cp1.wait()
        cp2.wait()

        def row(r, carry):
            for cc in range(D_MODEL // 16):
                sl = pl.ds(cc * 16, 16)
                y1_v[r, sl] = y1_v[r, sl] + y2_v[r, sl]
            return carry

        lax.fori_loop(0, 64, row, 0)
        pltpu.sync_copy(y1_v, o_hbm.at[pl.ds(base, 64)])

    return dispatch, gather_rows, combine


def kernel(hidden_states, router_w, gate_w, up_w, down_w):
    bsz, seq_len, d_model = hidden_states.shape
    x = hidden_states.reshape(-1, d_model)

    sched = pl.pallas_call(
        _router_body,
        out_shape=jax.ShapeDtypeStruct((T, 4), jnp.float32),
    )(x, router_w)
    e1 = sched[:, 0].astype(jnp.int32)
    e2 = sched[:, 1].astype(jnp.int32)
    ef = jnp.concatenate([e1, e2])
    wf = jnp.concatenate([sched[:, 2], sched[:, 3]])

    ef_t = ef.reshape(16, 256).T
    rank_t, totb = pl.pallas_call(
        _rank_body,
        out_shape=(jax.ShapeDtypeStruct((256, 16), jnp.float32),
                   jax.ShapeDtypeStruct((8, 128), jnp.float32)),
    )(ef_t, jnp.asarray(_TRI), jnp.asarray(_GM), jnp.asarray(_SEL))
    rank = rank_t.T.reshape(A).astype(jnp.int32)
    counts = totb[0].reshape(E, 16).sum(axis=1).astype(jnp.int32)

    nblk = (counts + G - 1) // G
    ends = jnp.cumsum(nblk)
    starts = jnp.concatenate([jnp.zeros(1, jnp.int32), ends[:-1]])
    pb16 = jnp.concatenate([starts * G, jnp.zeros(8, jnp.int32)])
    bid = jnp.arange(NB, dtype=jnp.int32)
    block_expert = jnp.minimum(
        jnp.sum((bid[:, None] >= ends[None, :]).astype(jnp.int32), axis=1),
        E - 1).astype(jnp.int32)

    dispatch, gather_rows, combine = _sc_kernels()
    order, ws, dest = dispatch(ef, rank, pb16, wf,
                               jnp.zeros((PT,), jnp.int32),
                               jnp.zeros((PT,), jnp.float32))

    xs = gather_rows(x, order)

    y = pl.pallas_call(
        _ffn_body,
        grid_spec=pltpu.PrefetchScalarGridSpec(
            num_scalar_prefetch=1,
            grid=(NB,),
            in_specs=[
                pl.BlockSpec((G, D_MODEL), lambda b, be: (b, 0)),
                pl.BlockSpec((1, INTER, D_MODEL), lambda b, be: (be[b], 0, 0)),
                pl.BlockSpec((1, INTER, D_MODEL), lambda b, be: (be[b], 0, 0)),
                pl.BlockSpec((1, D_MODEL, INTER), lambda b, be: (be[b], 0, 0)),
                pl.BlockSpec((G, 1), lambda b, be: (b, 0)),
            ],
            out_specs=pl.BlockSpec((G, D_MODEL), lambda b, be: (b, 0)),
        ),
        out_shape=jax.ShapeDtypeStruct((PT, D_MODEL), jnp.float32),
        compiler_params=pltpu.CompilerParams(
            dimension_semantics=("arbitrary",)),
    )(block_expert, xs, gate_w, up_w, down_w, ws.reshape(PT, 1))

    out = combine(y, dest[:T], dest[T:])
    return out.reshape(bsz, seq_len, d_model)
